# baseline (device time: 9918 ns/iter reference)
import jax
import jax.numpy as jnp
from jax import lax
from jax.experimental import pallas as pl
from jax.experimental.pallas import tpu as pltpu

N_DEV = 8


def kernel(x):
    m_per, n = x.shape

    def body(x_ref, out_ref, part_ref, pairin_ref, pairpart_ref, gather_ref,
             p1_send_sem, p1_recv_sem, p2_send_sems, p2_recv_sems):
        my_pos = lax.axis_index("i")
        q = lax.rem(my_pos, 4)
        zbase = my_pos - q
        partner = lax.rem(my_pos + 4, 8)

        barrier_sem = pltpu.get_barrier_semaphore()
        pl.semaphore_signal(barrier_sem, inc=1, device_id=(partner,),
                            device_id_type=pl.DeviceIdType.MESH)
        for r in range(4):
            @pl.when(r != q)
            def _():
                pl.semaphore_signal(barrier_sem, inc=1,
                                    device_id=(zbase + r,),
                                    device_id_type=pl.DeviceIdType.MESH)

        xv = x_ref[:, :].astype(jnp.float32)
        mx = jnp.max(xv, axis=0, keepdims=True)
        first = jnp.argmax(xv, axis=0).astype(jnp.int32)[None, :]
        gidx = (first + my_pos * m_per).astype(jnp.float32)
        part_ref[0:1, :] = mx
        part_ref[1:2, :] = gidx

        pl.semaphore_wait(barrier_sem, 4)

        p1 = pltpu.make_async_remote_copy(
            src_ref=part_ref, dst_ref=pairin_ref,
            send_sem=p1_send_sem, recv_sem=p1_recv_sem,
            device_id=(partner,), device_id_type=pl.DeviceIdType.MESH,
        )
        p1.start()
        p1.wait()

        v0, i0 = part_ref[0:1, :], part_ref[1:2, :]
        v1, i1 = pairin_ref[0:1, :], pairin_ref[1:2, :]
        pv = jnp.maximum(v0, v1)
        pi = jnp.where(
            v0 == v1, jnp.minimum(i0, i1), jnp.where(v0 > v1, i0, i1)
        )
        pairpart_ref[0:1, :] = pv
        pairpart_ref[1:2, :] = pi
        gather_ref[pl.ds(q, 1), :, :] = pairpart_ref[:, :][None, :, :]

        for r in range(4):
            @pl.when(r != q)
            def _():
                rdma = pltpu.make_async_remote_copy(
                    src_ref=pairpart_ref,
                    dst_ref=gather_ref.at[q],
                    send_sem=p2_send_sems.at[r],
                    recv_sem=p2_recv_sems.at[q],
                    device_id=(zbase + r,),
                    device_id_type=pl.DeviceIdType.MESH,
                )
                rdma.start()
        for r in range(4):
            @pl.when(r != q)
            def _():
                d = pltpu.make_async_remote_copy(
                    src_ref=pairpart_ref,
                    dst_ref=gather_ref.at[r],
                    send_sem=p2_send_sems.at[r],
                    recv_sem=p2_recv_sems.at[r],
                    device_id=(zbase + r,),
                    device_id_type=pl.DeviceIdType.MESH,
                )
                d.wait_recv()
                d.wait_send()

        vals = gather_ref[:, 0, :]
        idxs = gather_ref[:, 1, :]
        gmax = jnp.max(vals, axis=0, keepdims=True)
        gidx_out = jnp.min(
            jnp.where(vals == gmax, idxs, jnp.float32(1e9)),
            axis=0, keepdims=True,
        )
        out_ref[0:1, :] = gmax
        out_ref[1:2, :] = gidx_out

    return pl.pallas_call(
        body,
        out_shape=jax.ShapeDtypeStruct((2, n), jnp.float32),
        in_specs=[pl.BlockSpec(memory_space=pltpu.VMEM)],
        out_specs=pl.BlockSpec(memory_space=pltpu.VMEM),
        scratch_shapes=[
            pltpu.VMEM((2, n), jnp.float32),
            pltpu.VMEM((2, n), jnp.float32),
            pltpu.VMEM((2, n), jnp.float32),
            pltpu.VMEM((4, 2, n), jnp.float32),
            pltpu.SemaphoreType.DMA,
            pltpu.SemaphoreType.DMA,
            pltpu.SemaphoreType.DMA((4,)),
            pltpu.SemaphoreType.DMA((4,)),
        ],
        compiler_params=pltpu.CompilerParams(collective_id=0),
    )(x)


# device time: 8760 ns/iter; 1.1322x vs baseline; 1.1322x over previous
import jax
import jax.numpy as jnp
from jax import lax
from jax.experimental import pallas as pl
from jax.experimental.pallas import tpu as pltpu

N_DEV = 8


def kernel(x):
    m_per, n = x.shape

    def body(x_ref, out_ref, part_ref, gather_ref, send_sems, recv_sems):
        my_pos = lax.axis_index("i")

        barrier_sem = pltpu.get_barrier_semaphore()
        for j in range(N_DEV):
            @pl.when(j != my_pos)
            def _():
                pl.semaphore_signal(
                    barrier_sem, inc=1,
                    device_id=(j,), device_id_type=pl.DeviceIdType.MESH,
                )

        xv = x_ref[:, :].astype(jnp.float32)
        mx = jnp.max(xv, axis=0, keepdims=True)
        first = jnp.argmax(xv, axis=0).astype(jnp.int32)[None, :]
        gidx = (first + my_pos * m_per).astype(jnp.float32)
        part_ref[0:1, :] = mx
        part_ref[1:2, :] = gidx

        gather_ref[pl.ds(my_pos, 1), :, :] = part_ref[:, :][None, :, :]

        pl.semaphore_wait(barrier_sem, N_DEV - 1)

        for j in range(N_DEV):
            @pl.when(j != my_pos)
            def _():
                rdma = pltpu.make_async_remote_copy(
                    src_ref=part_ref,
                    dst_ref=gather_ref.at[my_pos],
                    send_sem=send_sems.at[j],
                    recv_sem=recv_sems.at[my_pos],
                    device_id=(j,),
                    device_id_type=pl.DeviceIdType.MESH,
                )
                rdma.start()

        for j in range(N_DEV):
            @pl.when(j != my_pos)
            def _():
                d = pltpu.make_async_remote_copy(
                    src_ref=part_ref,
                    dst_ref=gather_ref.at[j],
                    send_sem=send_sems.at[j],
                    recv_sem=recv_sems.at[j],
                    device_id=(j,),
                    device_id_type=pl.DeviceIdType.MESH,
                )
                d.wait_recv()
                d.wait_send()

        vals = gather_ref[:, 0, :]
        idxs = gather_ref[:, 1, :]
        gmax = jnp.max(vals, axis=0, keepdims=True)
        gidx_out = jnp.min(
            jnp.where(vals == gmax, idxs, jnp.float32(1e9)),
            axis=0, keepdims=True,
        )
        out_ref[0:1, :] = gmax
        out_ref[1:2, :] = gidx_out

    return pl.pallas_call(
        body,
        out_shape=jax.ShapeDtypeStruct((2, n), jnp.float32),
        in_specs=[pl.BlockSpec(memory_space=pltpu.VMEM)],
        out_specs=pl.BlockSpec(memory_space=pltpu.VMEM),
        scratch_shapes=[
            pltpu.VMEM((2, n), jnp.float32),
            pltpu.VMEM((N_DEV, 2, n), jnp.float32),
            pltpu.SemaphoreType.DMA((N_DEV,)),
            pltpu.SemaphoreType.DMA((N_DEV,)),
        ],
        compiler_params=pltpu.CompilerParams(collective_id=0),
    )(x)
